# Initial kernel scaffold; baseline (speedup 1.0000x reference)
import jax
import jax.numpy as jnp
from jax.experimental import pallas as pl


def kernel(cell_feat, net_feat, pin_feat, pins_src, pins_dst, pt_src, pt_dst, W_cell, b_cell, W_net, b_net, W_pin, b_pin, W_gc, b_gc, We1, be1, We2, be2, Wn, bn, Wo, bo, W_dis, b_dis, W_ang, b_ang):
    raise NotImplementedError("write your pallas kernel here")


# speedup vs baseline: 2.1126x; 2.1126x over previous
import functools

import jax
import jax.numpy as jnp
from jax import lax
from jax.experimental import pallas as pl
from jax.experimental.pallas import tpu as pltpu
from jax.experimental.pallas import tpu_sc as plsc

N_CELL = 50000
N_NET = 10000
N_PIN = 800000
N_PT = 800000
HC = 64
HH = HC // 2
NC, NS, LANES = 2, 16, 16
K = 128
_LN2 = 0.6931471805599453


def _ssp(x):
    return jnp.maximum(x, 0.0) + jnp.log1p(jnp.exp(-jnp.abs(x))) - _LN2


_BP = 6400


def _pin_body(x_ref, wp_ref, bp_ref, w1_ref, b1_ref, w2_ref, b2_ref, out_ref):
    x = x_ref[...]
    hp = jnp.tanh(jnp.dot(x, wp_ref[...], preferred_element_type=jnp.float32)
                  + bp_ref[...])
    t = _ssp(jnp.dot(hp, w1_ref[...], preferred_element_type=jnp.float32)
             + b1_ref[...])
    he = _ssp(jnp.dot(t, w2_ref[...], preferred_element_type=jnp.float32)
              + b2_ref[...])
    out_ref[0] = he[:, :HH]
    out_ref[1] = he[:, HH:]


def _pin_mlp(pin_feat, W_pin, b_pin, We1, be1, We2, be2):
    rp, hp = W_pin.shape
    return pl.pallas_call(
        _pin_body,
        grid=(N_PIN // _BP,),
        in_specs=[
            pl.BlockSpec((_BP, rp), lambda i: (i, 0)),
            pl.BlockSpec((rp, hp), lambda i: (0, 0)),
            pl.BlockSpec((hp,), lambda i: (0,)),
            pl.BlockSpec((hp, HC), lambda i: (0, 0)),
            pl.BlockSpec((HC,), lambda i: (0,)),
            pl.BlockSpec((HC, HC), lambda i: (0, 0)),
            pl.BlockSpec((HC,), lambda i: (0,)),
        ],
        out_specs=pl.BlockSpec((2, _BP, HH), lambda i: (0, i, 0)),
        out_shape=jax.ShapeDtypeStruct((2, N_PIN, HH), jnp.float32),
    )(pin_feat, W_pin, b_pin, We1, be1, We2, be2)


def _net_body(nf_ref, wn_ref, bn_ref, wv_ref, bv_ref, out_ref):
    hn = jnp.tanh(jnp.dot(nf_ref[...], wn_ref[...],
                          preferred_element_type=jnp.float32) + bn_ref[...])
    hv = jnp.dot(hn, wv_ref[...], preferred_element_type=jnp.float32) + bv_ref[...]
    out_ref[0] = hv[:, :HH]
    out_ref[1] = hv[:, HH:]


def _net_mlp(net_feat, W_net, b_net, Wn, bn):
    return pl.pallas_call(
        _net_body,
        out_shape=jax.ShapeDtypeStruct((2, N_NET, HH), jnp.float32),
    )(net_feat, W_net, b_net, Wn, bn)


_BC = 5000


def _cell_body(agg_ref, wo_ref, bo_ref, wcat_ref, bvec_ref, p_ref):
    ncell = _ssp(
        jnp.dot(agg_ref[0], wo_ref[...][:HH], preferred_element_type=jnp.float32)
        + jnp.dot(agg_ref[1], wo_ref[...][HH:], preferred_element_type=jnp.float32)
        + bo_ref[...])
    p_ref[...] = (jnp.dot(ncell, wcat_ref[...],
                          preferred_element_type=jnp.float32) + bvec_ref[...])


def _cell_mlp(agg2, Wo, bo, Wcat, bvec):
    return pl.pallas_call(
        _cell_body,
        grid=(N_CELL // _BC,),
        in_specs=[
            pl.BlockSpec((2, _BC, HH), lambda i: (0, i, 0)),
            pl.BlockSpec((HC, HC), lambda i: (0, 0)),
            pl.BlockSpec((HC,), lambda i: (0,)),
            pl.BlockSpec((HC, 16), lambda i: (0, 0)),
            pl.BlockSpec((16,), lambda i: (0,)),
        ],
        out_specs=pl.BlockSpec((_BC, 16), lambda i: (i, 0)),
        out_shape=jax.ShapeDtypeStruct((N_CELL, 16), jnp.float32),
    )(agg2, Wo, bo, Wcat, bvec)


_mesh = plsc.VectorSubcoreMesh(core_axis_name="c", subcore_axis_name="s",
                               num_cores=NC, num_subcores=NS)
_ZR = 1000
_NCH_PIN = N_PIN // K


def _rr_loop(n_chunks, sid, body):
    n_g = n_chunks // NS + jnp.where(sid < n_chunks % NS, 1, 0)
    lax.fori_loop(0, n_g, lambda g, c: body(g * NS + sid) or c, 0)


KA = 64
_NCH_A = N_PIN // KA


@functools.partial(
    pl.kernel,
    out_type=jax.ShapeDtypeStruct((NC, N_CELL, HH), jnp.float32),
    mesh=_mesh,
    scratch_types=[
        pltpu.VMEM_SHARED((N_CELL, HH), jnp.float32),
        pltpu.VMEM((KA,), jnp.int32),
        pltpu.VMEM((KA,), jnp.int32),
        pltpu.VMEM((KA, HH), jnp.float32),
        pltpu.VMEM((KA, HH), jnp.float32),
        pltpu.SemaphoreType.DMA,
    ],
    compiler_params=pltpu.CompilerParams(use_tc_tiling_on_sc=False),
)
def _agg_kernel(ps_hbm, pd_hbm, he_hbm, hv_hbm, zeros_hbm, out_hbm,
                acc_sh, idx_s, idx_d, he_v, hvr_v, sem):
    cid = lax.axis_index("c")
    sid = lax.axis_index("s")

    def zchunk(c):
        off = pl.multiple_of(c * _ZR, 8)
        pltpu.sync_copy(zeros_hbm, acc_sh.at[pl.ds(off, _ZR)])

    _rr_loop(N_CELL // _ZR, sid, zchunk)
    plsc.subcore_barrier()

    n_g = (_NCH_A // NS) + jnp.where(sid < _NCH_A % NS, 1, 0)

    def chunk(g, c):
        base = pl.multiple_of((g * NS + sid) * KA, KA)
        pltpu.sync_copy(ps_hbm.at[pl.ds(base, KA)], idx_s)
        pltpu.sync_copy(pd_hbm.at[pl.ds(base, KA)], idx_d)
        pltpu.sync_copy(he_hbm.at[cid, pl.ds(base, KA)], he_v)

        pltpu.async_copy(hv_hbm.at[cid].at[idx_d], hvr_v, sem).wait()

        def mrow(i, cc):
            he_v[i, pl.ds(0, LANES)] = (he_v[i, pl.ds(0, LANES)]
                                        * hvr_v[i, pl.ds(0, LANES)])
            he_v[i, pl.ds(LANES, LANES)] = (he_v[i, pl.ds(LANES, LANES)]
                                            * hvr_v[i, pl.ds(LANES, LANES)])
            return cc

        lax.fori_loop(0, KA, mrow, 0)
        pltpu.sync_copy(he_v, acc_sh.at[idx_s], add=True)
        return c

    lax.fori_loop(0, n_g, chunk, 0)
    plsc.subcore_barrier()

    def ochunk(c):
        off = pl.multiple_of(c * _ZR, 8)
        pltpu.sync_copy(acc_sh.at[pl.ds(off, _ZR)],
                        out_hbm.at[cid, pl.ds(off, _ZR)])

    _rr_loop(N_CELL // _ZR, sid, ochunk)


_NCH_PT = N_PT // K


@functools.partial(
    pl.kernel,
    out_type=(jax.ShapeDtypeStruct((N_PT,), jnp.float32),
              jax.ShapeDtypeStruct((N_PT,), jnp.float32)),
    mesh=_mesh,
    scratch_types=[
        pltpu.VMEM_SHARED((N_CELL,), jnp.float32),
        pltpu.VMEM_SHARED((N_CELL,), jnp.float32),
        pltpu.VMEM_SHARED((N_CELL,), jnp.float32),
        pltpu.VMEM_SHARED((N_CELL,), jnp.float32),
        pltpu.VMEM((K,), jnp.int32),
        pltpu.VMEM((K,), jnp.int32),
        pltpu.VMEM((K,), jnp.float32),
        pltpu.VMEM((K,), jnp.float32),
        pltpu.VMEM((K,), jnp.float32),
        pltpu.VMEM((K,), jnp.float32),
        pltpu.VMEM((K,), jnp.float32),
        pltpu.VMEM((K,), jnp.float32),
        pltpu.SemaphoreType.DMA,
    ],
    compiler_params=pltpu.CompilerParams(use_tc_tiling_on_sc=False),
)
def _readout_kernel(src_hbm, dst_hbm, p0_hbm, p1_hbm, p2_hbm, p3_hbm,
                    dis_hbm, ang_hbm,
                    p0_sh, p1_sh, p2_sh, p3_sh, idx_s, idx_d,
                    s0_v, s1_v, d2_v, d3_v, dis_v, ang_v, sem):
    cid = lax.axis_index("c")
    sid = lax.axis_index("s")
    wid = sid * NC + cid

    def pchunk(c):
        off = pl.multiple_of(c * _ZR, 8)
        sl = pl.ds(off, _ZR)
        pltpu.sync_copy(p0_hbm.at[sl], p0_sh.at[sl])
        pltpu.sync_copy(p1_hbm.at[sl], p1_sh.at[sl])
        pltpu.sync_copy(p2_hbm.at[sl], p2_sh.at[sl])
        pltpu.sync_copy(p3_hbm.at[sl], p3_sh.at[sl])

    _rr_loop(N_CELL // _ZR, sid, pchunk)
    plsc.subcore_barrier()

    nw = NC * NS
    n_g = (_NCH_PT // nw) + jnp.where(wid < _NCH_PT % nw, 1, 0)

    def chunk(g, c):
        base = pl.multiple_of((g * nw + wid) * K, K)
        pltpu.sync_copy(src_hbm.at[pl.ds(base, K)], idx_s)
        pltpu.sync_copy(dst_hbm.at[pl.ds(base, K)], idx_d)
        cp0 = pltpu.async_copy(p0_sh.at[idx_s], s0_v, sem)
        cp1 = pltpu.async_copy(p1_sh.at[idx_s], s1_v, sem)
        cp2 = pltpu.async_copy(p2_sh.at[idx_d], d2_v, sem)
        cp3 = pltpu.async_copy(p3_sh.at[idx_d], d3_v, sem)
        cp0.wait()
        cp1.wait()
        cp2.wait()
        cp3.wait()
        for j in range(K // LANES):
            sl = pl.ds(j * LANES, LANES)
            dis_v[sl] = jnp.exp(s0_v[sl] + d2_v[sl])
            y = s1_v[sl] + d3_v[sl]
            e = jnp.exp(-2.0 * jnp.abs(y))
            t = (1.0 - e) / (1.0 + e)
            ang_v[sl] = jnp.where(y < 0.0, -4.0, 4.0) * t
        pltpu.sync_copy(dis_v, dis_hbm.at[pl.ds(base, K)])
        pltpu.sync_copy(ang_v, ang_hbm.at[pl.ds(base, K)])
        return c

    lax.fori_loop(0, n_g, chunk, 0)


def kernel(cell_feat, net_feat, pin_feat, pins_src, pins_dst, pt_src, pt_dst,
           W_cell, b_cell, W_net, b_net, W_pin, b_pin, W_gc, b_gc,
           We1, be1, We2, be2, Wn, bn, Wo, bo, W_dis, b_dis, W_ang, b_ang):
    he2 = _pin_mlp(pin_feat, W_pin, b_pin, We1, be1, We2, be2)
    hv2 = _net_mlp(net_feat, W_net, b_net, Wn, bn)
    zeros = jnp.zeros((_ZR, HH), jnp.float32)
    agg2 = _agg_kernel(pins_src, pins_dst, he2, hv2, zeros)
    Wcat = (jnp.zeros((HC, 16), jnp.float32)
            .at[:, 0].set(W_dis[:HC, 0]).at[:, 1].set(W_ang[:HC, 0])
            .at[:, 2].set(W_dis[HC:, 0]).at[:, 3].set(W_ang[HC:, 0]))
    bvec = jnp.zeros((16,), jnp.float32).at[0].set(b_dis[0]).at[1].set(b_ang[0])
    P = _cell_mlp(agg2, Wo, bo, Wcat, bvec)
    p0, p1, p2, p3 = P[:, 0], P[:, 1], P[:, 2], P[:, 3]
    edge_dis, edge_angle = _readout_kernel(pt_src, pt_dst, p0, p1, p2, p3)
    return (edge_dis, edge_angle)


# speedup vs baseline: 2.8264x; 1.3379x over previous
import functools

import jax
import jax.numpy as jnp
from jax import lax
from jax.experimental import pallas as pl
from jax.experimental.pallas import tpu as pltpu
from jax.experimental.pallas import tpu_sc as plsc

N_CELL = 50000
N_NET = 10000
N_PIN = 800000
N_PT = 800000
HC = 64
HH = HC // 2
NC, NS, LANES = 2, 16, 16
K = 128
_LN2 = 0.6931471805599453


def _ssp(x):
    return jnp.maximum(x, 0.0) + jnp.log1p(jnp.exp(-jnp.abs(x))) - _LN2


_BP = 6400


def _pin_body(x_ref, wp_ref, bp_ref, w1_ref, b1_ref, w2_ref, b2_ref, out_ref):
    x = x_ref[...]
    hp = jnp.tanh(jnp.dot(x, wp_ref[...], preferred_element_type=jnp.float32)
                  + bp_ref[...])
    t = _ssp(jnp.dot(hp, w1_ref[...], preferred_element_type=jnp.float32)
             + b1_ref[...])
    he = _ssp(jnp.dot(t, w2_ref[...], preferred_element_type=jnp.float32)
              + b2_ref[...])
    out_ref[0] = he[:, :HH]
    out_ref[1] = he[:, HH:]


def _pin_mlp(pin_feat, W_pin, b_pin, We1, be1, We2, be2):
    rp, hp = W_pin.shape
    return pl.pallas_call(
        _pin_body,
        grid=(N_PIN // _BP,),
        in_specs=[
            pl.BlockSpec((_BP, rp), lambda i: (i, 0)),
            pl.BlockSpec((rp, hp), lambda i: (0, 0)),
            pl.BlockSpec((hp,), lambda i: (0,)),
            pl.BlockSpec((hp, HC), lambda i: (0, 0)),
            pl.BlockSpec((HC,), lambda i: (0,)),
            pl.BlockSpec((HC, HC), lambda i: (0, 0)),
            pl.BlockSpec((HC,), lambda i: (0,)),
        ],
        out_specs=pl.BlockSpec((2, _BP, HH), lambda i: (0, i, 0)),
        out_shape=jax.ShapeDtypeStruct((2, N_PIN, HH), jnp.float32),
    )(pin_feat, W_pin, b_pin, We1, be1, We2, be2)


def _net_body(nf_ref, wn_ref, bn_ref, wv_ref, bv_ref, out_ref):
    hn = jnp.tanh(jnp.dot(nf_ref[...], wn_ref[...],
                          preferred_element_type=jnp.float32) + bn_ref[...])
    hv = jnp.dot(hn, wv_ref[...], preferred_element_type=jnp.float32) + bv_ref[...]
    out_ref[0] = hv[:, :HH]
    out_ref[1] = hv[:, HH:]


def _net_mlp(net_feat, W_net, b_net, Wn, bn):
    return pl.pallas_call(
        _net_body,
        out_shape=jax.ShapeDtypeStruct((2, N_NET, HH), jnp.float32),
    )(net_feat, W_net, b_net, Wn, bn)


_BC = 5000


def _cell_body(agg_ref, wo_ref, bo_ref, wcat_ref, bvec_ref, p_ref):
    ncell = _ssp(
        jnp.dot(agg_ref[0], wo_ref[...][:HH], preferred_element_type=jnp.float32)
        + jnp.dot(agg_ref[1], wo_ref[...][HH:], preferred_element_type=jnp.float32)
        + bo_ref[...])
    p_ref[...] = (jnp.dot(ncell, wcat_ref[...],
                          preferred_element_type=jnp.float32) + bvec_ref[...])


def _cell_mlp(agg2, Wo, bo, Wcat, bvec):
    return pl.pallas_call(
        _cell_body,
        grid=(N_CELL // _BC,),
        in_specs=[
            pl.BlockSpec((2, _BC, HH), lambda i: (0, i, 0)),
            pl.BlockSpec((HC, HC), lambda i: (0, 0)),
            pl.BlockSpec((HC,), lambda i: (0,)),
            pl.BlockSpec((HC, 16), lambda i: (0, 0)),
            pl.BlockSpec((16,), lambda i: (0,)),
        ],
        out_specs=pl.BlockSpec((_BC, 16), lambda i: (i, 0)),
        out_shape=jax.ShapeDtypeStruct((N_CELL, 16), jnp.float32),
    )(agg2, Wo, bo, Wcat, bvec)


_mesh = plsc.VectorSubcoreMesh(core_axis_name="c", subcore_axis_name="s",
                               num_cores=NC, num_subcores=NS)
_ZR = 1000
_NCH_PIN = N_PIN // K


def _rr_loop(n_chunks, sid, body):
    n_g = n_chunks // NS + jnp.where(sid < n_chunks % NS, 1, 0)
    lax.fori_loop(0, n_g, lambda g, c: body(g * NS + sid) or c, 0)


KA = 128
_NCH_A = N_PIN // KA


@functools.partial(
    pl.kernel,
    out_type=jax.ShapeDtypeStruct((NC, N_CELL, HH), jnp.float32),
    mesh=_mesh,
    scratch_types=[
        pltpu.VMEM_SHARED((N_CELL, HH), jnp.float32),
        pltpu.VMEM_SHARED((N_NET, HH), jnp.float32),
        pltpu.VMEM((KA,), jnp.int32),
        pltpu.VMEM((KA,), jnp.int32),
        pltpu.VMEM((KA, HH), jnp.float32),
        pltpu.VMEM((KA, HH), jnp.float32),
        pltpu.SemaphoreType.DMA,
    ],
    compiler_params=pltpu.CompilerParams(use_tc_tiling_on_sc=False),
)
def _agg_kernel(ps_hbm, pd_hbm, he_hbm, hv_hbm, zeros_hbm, out_hbm,
                acc_sh, hv_sh, idx_s, idx_d, he_v, hvr_v, sem):
    cid = lax.axis_index("c")
    sid = lax.axis_index("s")

    def zchunk(c):
        off = pl.multiple_of(c * _ZR, 8)
        pltpu.sync_copy(zeros_hbm, acc_sh.at[pl.ds(off, _ZR)])

    _rr_loop(N_CELL // _ZR, sid, zchunk)

    def hvchunk(c):
        off = pl.multiple_of(c * _ZR, 8)
        pltpu.sync_copy(hv_hbm.at[cid, pl.ds(off, _ZR)], hv_sh.at[pl.ds(off, _ZR)])

    _rr_loop(N_NET // _ZR, sid, hvchunk)
    plsc.subcore_barrier()

    n_g = (_NCH_A // NS) + jnp.where(sid < _NCH_A % NS, 1, 0)

    def chunk(g, c):
        base = pl.multiple_of((g * NS + sid) * KA, KA)
        pltpu.sync_copy(ps_hbm.at[pl.ds(base, KA)], idx_s)
        pltpu.sync_copy(pd_hbm.at[pl.ds(base, KA)], idx_d)
        pltpu.sync_copy(he_hbm.at[cid, pl.ds(base, KA)], he_v)

        pltpu.async_copy(hv_sh.at[idx_d], hvr_v, sem).wait()

        def mrow(i, cc):
            he_v[i, pl.ds(0, LANES)] = (he_v[i, pl.ds(0, LANES)]
                                        * hvr_v[i, pl.ds(0, LANES)])
            he_v[i, pl.ds(LANES, LANES)] = (he_v[i, pl.ds(LANES, LANES)]
                                            * hvr_v[i, pl.ds(LANES, LANES)])
            return cc

        lax.fori_loop(0, KA, mrow, 0)
        pltpu.sync_copy(he_v, acc_sh.at[idx_s], add=True)
        return c

    lax.fori_loop(0, n_g, chunk, 0)
    plsc.subcore_barrier()

    def ochunk(c):
        off = pl.multiple_of(c * _ZR, 8)
        pltpu.sync_copy(acc_sh.at[pl.ds(off, _ZR)],
                        out_hbm.at[cid, pl.ds(off, _ZR)])

    _rr_loop(N_CELL // _ZR, sid, ochunk)


_NCH_PT = N_PT // K


@functools.partial(
    pl.kernel,
    out_type=(jax.ShapeDtypeStruct((N_PT,), jnp.float32),
              jax.ShapeDtypeStruct((N_PT,), jnp.float32)),
    mesh=_mesh,
    scratch_types=[
        pltpu.VMEM_SHARED((N_CELL,), jnp.float32),
        pltpu.VMEM_SHARED((N_CELL,), jnp.float32),
        pltpu.VMEM_SHARED((N_CELL,), jnp.float32),
        pltpu.VMEM_SHARED((N_CELL,), jnp.float32),
        pltpu.VMEM((K,), jnp.int32),
        pltpu.VMEM((K,), jnp.int32),
        pltpu.VMEM((K,), jnp.float32),
        pltpu.VMEM((K,), jnp.float32),
        pltpu.VMEM((K,), jnp.float32),
        pltpu.VMEM((K,), jnp.float32),
        pltpu.VMEM((K,), jnp.float32),
        pltpu.VMEM((K,), jnp.float32),
        pltpu.SemaphoreType.DMA,
    ],
    compiler_params=pltpu.CompilerParams(use_tc_tiling_on_sc=False),
)
def _readout_kernel(src_hbm, dst_hbm, p0_hbm, p1_hbm, p2_hbm, p3_hbm,
                    dis_hbm, ang_hbm,
                    p0_sh, p1_sh, p2_sh, p3_sh, idx_s, idx_d,
                    s0_v, s1_v, d2_v, d3_v, dis_v, ang_v, sem):
    cid = lax.axis_index("c")
    sid = lax.axis_index("s")
    wid = sid * NC + cid

    def pchunk(c):
        off = pl.multiple_of(c * _ZR, 8)
        sl = pl.ds(off, _ZR)
        pltpu.sync_copy(p0_hbm.at[sl], p0_sh.at[sl])
        pltpu.sync_copy(p1_hbm.at[sl], p1_sh.at[sl])
        pltpu.sync_copy(p2_hbm.at[sl], p2_sh.at[sl])
        pltpu.sync_copy(p3_hbm.at[sl], p3_sh.at[sl])

    _rr_loop(N_CELL // _ZR, sid, pchunk)
    plsc.subcore_barrier()

    nw = NC * NS
    n_g = (_NCH_PT // nw) + jnp.where(wid < _NCH_PT % nw, 1, 0)

    def chunk(g, c):
        base = pl.multiple_of((g * nw + wid) * K, K)
        pltpu.sync_copy(src_hbm.at[pl.ds(base, K)], idx_s)
        pltpu.sync_copy(dst_hbm.at[pl.ds(base, K)], idx_d)
        cp0 = pltpu.async_copy(p0_sh.at[idx_s], s0_v, sem)
        cp1 = pltpu.async_copy(p1_sh.at[idx_s], s1_v, sem)
        cp2 = pltpu.async_copy(p2_sh.at[idx_d], d2_v, sem)
        cp3 = pltpu.async_copy(p3_sh.at[idx_d], d3_v, sem)
        cp0.wait()
        cp1.wait()
        cp2.wait()
        cp3.wait()
        for j in range(K // LANES):
            sl = pl.ds(j * LANES, LANES)
            dis_v[sl] = jnp.exp(s0_v[sl] + d2_v[sl])
            y = s1_v[sl] + d3_v[sl]
            e = jnp.exp(-2.0 * jnp.abs(y))
            t = (1.0 - e) / (1.0 + e)
            ang_v[sl] = jnp.where(y < 0.0, -4.0, 4.0) * t
        pltpu.sync_copy(dis_v, dis_hbm.at[pl.ds(base, K)])
        pltpu.sync_copy(ang_v, ang_hbm.at[pl.ds(base, K)])
        return c

    lax.fori_loop(0, n_g, chunk, 0)


def kernel(cell_feat, net_feat, pin_feat, pins_src, pins_dst, pt_src, pt_dst,
           W_cell, b_cell, W_net, b_net, W_pin, b_pin, W_gc, b_gc,
           We1, be1, We2, be2, Wn, bn, Wo, bo, W_dis, b_dis, W_ang, b_ang):
    he2 = _pin_mlp(pin_feat, W_pin, b_pin, We1, be1, We2, be2)
    hv2 = _net_mlp(net_feat, W_net, b_net, Wn, bn)
    zeros = jnp.zeros((_ZR, HH), jnp.float32)
    agg2 = _agg_kernel(pins_src, pins_dst, he2, hv2, zeros)
    Wcat = (jnp.zeros((HC, 16), jnp.float32)
            .at[:, 0].set(W_dis[:HC, 0]).at[:, 1].set(W_ang[:HC, 0])
            .at[:, 2].set(W_dis[HC:, 0]).at[:, 3].set(W_ang[HC:, 0]))
    bvec = jnp.zeros((16,), jnp.float32).at[0].set(b_dis[0]).at[1].set(b_ang[0])
    P = _cell_mlp(agg2, Wo, bo, Wcat, bvec)
    p0, p1, p2, p3 = P[:, 0], P[:, 1], P[:, 2], P[:, 3]
    edge_dis, edge_angle = _readout_kernel(pt_src, pt_dst, p0, p1, p2, p3)
    return (edge_dis, edge_angle)


# speedup vs baseline: 3.4329x; 1.2146x over previous
import functools

import jax
import jax.numpy as jnp
from jax import lax
from jax.experimental import pallas as pl
from jax.experimental.pallas import tpu as pltpu
from jax.experimental.pallas import tpu_sc as plsc

N_CELL = 50000
N_NET = 10000
N_PIN = 800000
N_PT = 800000
HC = 64
HH = HC // 2
NC, NS, LANES = 2, 16, 16
K = 128
_LN2 = 0.6931471805599453


def _ssp(x):
    return jnp.maximum(x, 0.0) + jnp.log1p(jnp.exp(-jnp.abs(x))) - _LN2


_BP = 6400


def _pin_body(x_ref, wp_ref, bp_ref, w1_ref, b1_ref, w2_ref, b2_ref, out_ref):
    x = x_ref[...]
    hp = jnp.tanh(jnp.dot(x, wp_ref[...], preferred_element_type=jnp.float32)
                  + bp_ref[...])
    t = _ssp(jnp.dot(hp, w1_ref[...], preferred_element_type=jnp.float32)
             + b1_ref[...])
    he = _ssp(jnp.dot(t, w2_ref[...], preferred_element_type=jnp.float32)
              + b2_ref[...])
    out_ref[0] = he[:, :HH]
    out_ref[1] = he[:, HH:]


def _pin_mlp(pin_feat, W_pin, b_pin, We1, be1, We2, be2):
    rp, hp = W_pin.shape
    return pl.pallas_call(
        _pin_body,
        grid=(N_PIN // _BP,),
        in_specs=[
            pl.BlockSpec((_BP, rp), lambda i: (i, 0)),
            pl.BlockSpec((rp, hp), lambda i: (0, 0)),
            pl.BlockSpec((hp,), lambda i: (0,)),
            pl.BlockSpec((hp, HC), lambda i: (0, 0)),
            pl.BlockSpec((HC,), lambda i: (0,)),
            pl.BlockSpec((HC, HC), lambda i: (0, 0)),
            pl.BlockSpec((HC,), lambda i: (0,)),
        ],
        out_specs=pl.BlockSpec((2, _BP, HH), lambda i: (0, i, 0)),
        out_shape=jax.ShapeDtypeStruct((2, N_PIN, HH), jnp.float32),
    )(pin_feat, W_pin, b_pin, We1, be1, We2, be2)


def _net_body(nf_ref, wn_ref, bn_ref, wv_ref, bv_ref, out_ref):
    hn = jnp.tanh(jnp.dot(nf_ref[...], wn_ref[...],
                          preferred_element_type=jnp.float32) + bn_ref[...])
    hv = jnp.dot(hn, wv_ref[...], preferred_element_type=jnp.float32) + bv_ref[...]
    out_ref[0] = hv[:, :HH]
    out_ref[1] = hv[:, HH:]


def _net_mlp(net_feat, W_net, b_net, Wn, bn):
    return pl.pallas_call(
        _net_body,
        out_shape=jax.ShapeDtypeStruct((2, N_NET, HH), jnp.float32),
    )(net_feat, W_net, b_net, Wn, bn)


_BC = 5000


def _cell_body(agg_ref, wo_ref, bo_ref, wcat_ref, bvec_ref, p_ref):
    ncell = _ssp(
        jnp.dot(agg_ref[0], wo_ref[...][:HH], preferred_element_type=jnp.float32)
        + jnp.dot(agg_ref[1], wo_ref[...][HH:], preferred_element_type=jnp.float32)
        + bo_ref[...])
    p_ref[...] = (jnp.dot(ncell, wcat_ref[...],
                          preferred_element_type=jnp.float32) + bvec_ref[...])


def _cell_mlp(agg2, Wo, bo, Wcat, bvec):
    return pl.pallas_call(
        _cell_body,
        grid=(N_CELL // _BC,),
        in_specs=[
            pl.BlockSpec((2, _BC, HH), lambda i: (0, i, 0)),
            pl.BlockSpec((HC, HC), lambda i: (0, 0)),
            pl.BlockSpec((HC,), lambda i: (0,)),
            pl.BlockSpec((HC, 16), lambda i: (0, 0)),
            pl.BlockSpec((16,), lambda i: (0,)),
        ],
        out_specs=pl.BlockSpec((_BC, 16), lambda i: (i, 0)),
        out_shape=jax.ShapeDtypeStruct((N_CELL, 16), jnp.float32),
    )(agg2, Wo, bo, Wcat, bvec)


_mesh = plsc.VectorSubcoreMesh(core_axis_name="c", subcore_axis_name="s",
                               num_cores=NC, num_subcores=NS)
_ZR = 1000
_NCH_PIN = N_PIN // K


def _rr_loop(n_chunks, sid, body):
    n_g = n_chunks // NS + jnp.where(sid < n_chunks % NS, 1, 0)
    lax.fori_loop(0, n_g, lambda g, c: body(g * NS + sid) or c, 0)


KA = 128
SUP = 10
KS = SUP * KA
_NCH_A = N_PIN // KA


@functools.partial(
    pl.kernel,
    out_type=jax.ShapeDtypeStruct((NC, N_CELL, HH), jnp.float32),
    mesh=_mesh,
    scratch_types=[
        pltpu.VMEM_SHARED((N_CELL, HH), jnp.float32),
        pltpu.VMEM_SHARED((N_NET, HH), jnp.float32),
        pltpu.VMEM((2, KA), jnp.int32),
        pltpu.VMEM((KA, HH), jnp.float32),
        pltpu.VMEM((KA, HH), jnp.float32),
        pltpu.SemaphoreType.DMA,
    ],
    compiler_params=pltpu.CompilerParams(use_tc_tiling_on_sc=False),
)
def _agg_kernel(psd_hbm, he_hbm, hv_hbm, zeros_hbm, out_hbm,
                acc_sh, hv_sh, idx_v, he_v, hvr_v, sem):
    cid = lax.axis_index("c")
    sid = lax.axis_index("s")

    def zchunk(c):
        off = pl.multiple_of(c * _ZR, 8)
        pltpu.sync_copy(zeros_hbm, acc_sh.at[pl.ds(off, _ZR)])

    _rr_loop(N_CELL // _ZR, sid, zchunk)

    def hvchunk(c):
        off = pl.multiple_of(c * _ZR, 8)
        pltpu.sync_copy(hv_hbm.at[cid, pl.ds(off, _ZR)], hv_sh.at[pl.ds(off, _ZR)])

    _rr_loop(N_NET // _ZR, sid, hvchunk)
    plsc.subcore_barrier()

    def chunk(c):
        base = pl.multiple_of(c * KA, KA)
        ci = pltpu.async_copy(psd_hbm.at[c], idx_v, sem)
        ch = pltpu.async_copy(he_hbm.at[cid, pl.ds(base, KA)], he_v, sem)
        ci.wait()
        ch.wait()

        pltpu.async_copy(hv_sh.at[idx_v.at[1]], hvr_v, sem).wait()

        def mrow(i, cc):
            he_v[i, pl.ds(0, LANES)] = (he_v[i, pl.ds(0, LANES)]
                                        * hvr_v[i, pl.ds(0, LANES)])
            he_v[i, pl.ds(LANES, LANES)] = (he_v[i, pl.ds(LANES, LANES)]
                                            * hvr_v[i, pl.ds(LANES, LANES)])
            return cc

        lax.fori_loop(0, KA, mrow, 0)
        pltpu.sync_copy(he_v, acc_sh.at[idx_v.at[0]], add=True)

    _rr_loop(_NCH_A, sid, chunk)
    plsc.subcore_barrier()

    def ochunk(c):
        off = pl.multiple_of(c * _ZR, 8)
        pltpu.sync_copy(acc_sh.at[pl.ds(off, _ZR)],
                        out_hbm.at[cid, pl.ds(off, _ZR)])

    _rr_loop(N_CELL // _ZR, sid, ochunk)


_NSC_PT = N_PT // KS


@functools.partial(
    pl.kernel,
    out_type=(jax.ShapeDtypeStruct((N_PT,), jnp.float32),
              jax.ShapeDtypeStruct((N_PT,), jnp.float32)),
    mesh=_mesh,
    scratch_types=[
        pltpu.VMEM_SHARED((N_CELL,), jnp.float32),
        pltpu.VMEM_SHARED((N_CELL,), jnp.float32),
        pltpu.VMEM_SHARED((N_CELL,), jnp.float32),
        pltpu.VMEM_SHARED((N_CELL,), jnp.float32),
        pltpu.VMEM((SUP, KA), jnp.int32),
        pltpu.VMEM((SUP, KA), jnp.int32),
        pltpu.VMEM((KS,), jnp.float32),
        pltpu.VMEM((KS,), jnp.float32),
        pltpu.VMEM((KS,), jnp.float32),
        pltpu.VMEM((KS,), jnp.float32),
        pltpu.VMEM((KS,), jnp.float32),
        pltpu.VMEM((KS,), jnp.float32),
        pltpu.SemaphoreType.DMA,
    ],
    compiler_params=pltpu.CompilerParams(use_tc_tiling_on_sc=False),
)
def _readout_kernel(src_hbm, dst_hbm, p0_hbm, p1_hbm, p2_hbm, p3_hbm,
                    dis_hbm, ang_hbm,
                    p0_sh, p1_sh, p2_sh, p3_sh, idx_s, idx_d,
                    s0_v, s1_v, d2_v, d3_v, dis_v, ang_v, sem):
    cid = lax.axis_index("c")
    sid = lax.axis_index("s")
    wid = sid * NC + cid
    nw = NC * NS

    def pchunk(c):
        off = pl.multiple_of(c * _ZR, 8)
        sl = pl.ds(off, _ZR)
        pltpu.sync_copy(p0_hbm.at[sl], p0_sh.at[sl])
        pltpu.sync_copy(p1_hbm.at[sl], p1_sh.at[sl])
        pltpu.sync_copy(p2_hbm.at[sl], p2_sh.at[sl])
        pltpu.sync_copy(p3_hbm.at[sl], p3_sh.at[sl])

    _rr_loop(N_CELL // _ZR, sid, pchunk)
    plsc.subcore_barrier()

    n_g = (_NSC_PT // nw) + jnp.where(wid < _NSC_PT % nw, 1, 0)

    def chunk(g, c):
        sc = g * nw + wid
        row0 = sc * SUP
        base = pl.multiple_of(sc * KS, KS)
        pltpu.sync_copy(src_hbm.at[pl.ds(row0, SUP)], idx_s)
        pltpu.sync_copy(dst_hbm.at[pl.ds(row0, SUP)], idx_d)

        def bslab(b, cc):
            dst = pl.ds(pl.multiple_of(b * KA, KA), KA)
            cp0 = pltpu.async_copy(p0_sh.at[idx_s.at[b]], s0_v.at[dst], sem)
            cp1 = pltpu.async_copy(p1_sh.at[idx_s.at[b]], s1_v.at[dst], sem)
            cp2 = pltpu.async_copy(p2_sh.at[idx_d.at[b]], d2_v.at[dst], sem)
            cp3 = pltpu.async_copy(p3_sh.at[idx_d.at[b]], d3_v.at[dst], sem)
            cp0.wait()
            cp1.wait()
            cp2.wait()
            cp3.wait()
            for j in range(KA // LANES):
                sl = pl.ds(pl.multiple_of(b * KA + j * LANES, LANES), LANES)
                dis_v[sl] = jnp.exp(s0_v[sl] + d2_v[sl])
                y = s1_v[sl] + d3_v[sl]
                e = jnp.exp(-2.0 * jnp.abs(y))
                t = (1.0 - e) / (1.0 + e)
                ang_v[sl] = jnp.where(y < 0.0, -4.0, 4.0) * t
            return cc

        lax.fori_loop(0, SUP, bslab, 0)
        pltpu.sync_copy(dis_v, dis_hbm.at[pl.ds(base, KS)])
        pltpu.sync_copy(ang_v, ang_hbm.at[pl.ds(base, KS)])
        return c

    lax.fori_loop(0, n_g, chunk, 0)


def kernel(cell_feat, net_feat, pin_feat, pins_src, pins_dst, pt_src, pt_dst,
           W_cell, b_cell, W_net, b_net, W_pin, b_pin, W_gc, b_gc,
           We1, be1, We2, be2, Wn, bn, Wo, bo, W_dis, b_dis, W_ang, b_ang):
    he2 = _pin_mlp(pin_feat, W_pin, b_pin, We1, be1, We2, be2)
    hv2 = _net_mlp(net_feat, W_net, b_net, Wn, bn)
    zeros = jnp.zeros((_ZR, HH), jnp.float32)
    psd = jnp.stack([pins_src.reshape(N_PIN // KA, KA),
                     pins_dst.reshape(N_PIN // KA, KA)], axis=1)
    agg2 = _agg_kernel(psd, he2, hv2, zeros)
    Wcat = (jnp.zeros((HC, 16), jnp.float32)
            .at[:, 0].set(W_dis[:HC, 0]).at[:, 1].set(W_ang[:HC, 0])
            .at[:, 2].set(W_dis[HC:, 0]).at[:, 3].set(W_ang[HC:, 0]))
    bvec = jnp.zeros((16,), jnp.float32).at[0].set(b_dis[0]).at[1].set(b_ang[0])
    P = _cell_mlp(agg2, Wo, bo, Wcat, bvec)
    p0, p1, p2, p3 = P[:, 0], P[:, 1], P[:, 2], P[:, 3]
    edge_dis, edge_angle = _readout_kernel(pt_src.reshape(N_PT // KA, KA),
                                           pt_dst.reshape(N_PT // KA, KA),
                                           p0, p1, p2, p3)
    return (edge_dis, edge_angle)
